# Initial kernel scaffold; baseline (speedup 1.0000x reference)
import jax
import jax.numpy as jnp
from jax.experimental import pallas as pl


def kernel(x, edge_index, edge_attr, xA, noiselevel, distances, dosd_distances, batch, params):
    raise NotImplementedError("write your pallas kernel here")


# speedup vs baseline: 2.0242x; 2.0242x over previous
import functools

import jax
import jax.numpy as jnp
from jax.experimental import pallas as pl

N = 64
E = 1024
D = 584
NGFEAT = 21

INTERPRET = False


def _gnn_kernel(ei_r_ref, x_ref, eattr_ref, xA_ref, noise_ref, dosd_ref,
                *w_refs, out_ref):
    ws = [w[...] for w in w_refs]
    (n0w1, n0b1, n1w1, n1b1, lew1, leb1, gfw1, gfb1, ffw1, ffb1,
     n0w2, n0b2, n1w2, n1b2, lew2, leb2, gfw2, gfb2, ffw2, ffb2,
     n0w3, n0b3, n1w3, n1b3, lew3, leb3, gfw3, gfb3, ffw3, ffb3,
     nz0w, nz0b, nz1w, nz1b) = ws

    src_r = ei_r_ref[0:1, :]
    dst_r = ei_r_ref[1:2, :]
    rows_e = jax.lax.broadcasted_iota(jnp.int32, (E, 1), 0)
    cols_n = jax.lax.broadcasted_iota(jnp.int32, (1, N), 1)
    osrc = (jnp.broadcast_to(src_r.T, (E, 1)) == cols_n).astype(jnp.float32)
    odst = (jnp.broadcast_to(dst_r.T, (E, 1)) == cols_n).astype(jnp.float32)
    odst_t = (jax.lax.broadcasted_iota(jnp.int32, (N, 1), 0)
              == dst_r).astype(jnp.float32)
    del rows_e

    rowg = jnp.dot(osrc, dosd_ref[...], preferred_element_type=jnp.float32)
    dosd_vals = jnp.sum(rowg * odst, axis=1, keepdims=True)
    eattr18 = jnp.concatenate([eattr_ref[...], dosd_vals], axis=1)

    nz = noise_ref[0, 0]
    hnz = nz * nz0w[0, :] + nz0b[0, :]
    hnz = 0.5 * hnz * (1.0 + jax.lax.erf(hnz / jnp.sqrt(2.0).astype(jnp.float32)))
    noise_val = jnp.sum(hnz * nz1w[:, 0]) + nz1b[0, 0]

    h = jnp.concatenate(
        [x_ref[...], jnp.full((N, 1), noise_val, jnp.float32)], axis=1)

    def conv(h, n0w, n0b, n1w, n1b, lew, leb, gfw, gfb):
        el = jnp.dot(eattr18, lew, preferred_element_type=jnp.float32) + leb[0]
        hsrc = jnp.dot(osrc, h, preferred_element_type=jnp.float32)
        msg = jax.nn.relu(hsrc + el)
        aggr = jnp.dot(odst_t, msg, preferred_element_type=jnp.float32)
        gf = jnp.dot(xA_ref[...], gfw, preferred_element_type=jnp.float32) + gfb[0]
        out = aggr + h + gf
        t = jax.nn.relu(jnp.dot(out, n0w, preferred_element_type=jnp.float32) + n0b[0])
        return jnp.dot(t, n1w, preferred_element_type=jnp.float32) + n1b[0]

    h = jax.nn.relu(conv(h, n0w1, n0b1, n1w1, n1b1, lew1, leb1, gfw1, gfb1))
    h = jax.nn.relu(jnp.dot(h, ffw1, preferred_element_type=jnp.float32) + ffb1[0])
    h = jax.nn.relu(conv(h, n0w2, n0b2, n1w2, n1b2, lew2, leb2, gfw2, gfb2))
    h = jax.nn.relu(jnp.dot(h, ffw2, preferred_element_type=jnp.float32) + ffb2[0])
    h = jax.nn.relu(conv(h, n0w3, n0b3, n1w3, n1b3, lew3, leb3, gfw3, gfb3))
    out_ref[...] = jnp.dot(h, ffw3, preferred_element_type=jnp.float32) + ffb3[0]


def _eam_kernel(ei_r_ref, eattr_ref, out_ref):
    src_r = ei_r_ref[0:1, :]
    dst_r = ei_r_ref[1:2, :]
    key_r = src_r * N + dst_r
    key_c = key_r.T
    same = (key_c == key_r).astype(jnp.float32)
    later = (jax.lax.broadcasted_iota(jnp.int32, (E, 1), 0)
             > jax.lax.broadcasted_iota(jnp.int32, (1, E), 1)).astype(jnp.float32)
    conflict = jnp.max(same * later, axis=0, keepdims=True)
    winner = 1.0 - conflict
    keys4096 = jax.lax.broadcasted_iota(jnp.int32, (N * N, 1), 0)
    ot = (keys4096 == key_r).astype(jnp.float32) * winner
    out_ref[...] = jnp.dot(ot, eattr_ref[...], preferred_element_type=jnp.float32)


def _pair_kernel(pf_ref, *w_refs, out_ref):
    ws = [w[...] for w in w_refs]
    (bg0, bb0, bw00, bb00, bw01, bb01,
     bg1, bb1, bw10, bb10, bw11, bb11,
     mg0, mb0, mw00, mb00, mw01, mb01,
     mg1, mb1, mw10, mb10, mw11, mb11,
     lbg, lbb, lmg, lmb, redw, redb) = ws

    pf = pf_ref[...]

    def ln(a, g, b):
        m = jnp.mean(a, axis=-1, keepdims=True)
        v = jnp.mean((a - m) ** 2, axis=-1, keepdims=True)
        return (a - m) * jax.lax.rsqrt(v + 1e-5) * g[0] + b[0]

    def blk(a, g, b, w0, b0, w1, b1):
        t = ln(a, g, b)
        t = jax.nn.relu(jnp.dot(t, w0, preferred_element_type=jnp.float32) + b0[0])
        return jnp.dot(t, w1, preferred_element_type=jnp.float32) + b1[0] + a

    xb = blk(pf, bg0, bb0, bw00, bb00, bw01, bb01)
    xb = blk(xb, bg1, bb1, bw10, bb10, bw11, bb11)
    xb = ln(xb, lbg, lbb)
    xm = blk(pf, mg0, mb0, mw00, mb00, mw01, mb01)
    xm = blk(xm, mg1, mb1, mw10, mb10, mw11, mb11)
    xm = ln(xm, lmg, lmb)
    ob = jnp.dot(xb, redw[:, 0:1], preferred_element_type=jnp.float32)
    om = jnp.dot(xm, redw[:, 1:2], preferred_element_type=jnp.float32)
    out_ref[...] = jnp.concatenate([ob, om], axis=1) + redb[0]


def _quad_kernel(pbs_blk_ref, pms_blk_ref, pbs_ref, pms_ref, out_ref):
    val = (pbs_blk_ref[...][:, :, None, None]
           + pbs_ref[...][None, None, :, :]
           + pms_blk_ref[...][:, None, :, None]
           + pms_ref[...][None, :, None, :])
    out_ref[...] = jax.nn.sigmoid(val)


def _full(shape):
    return pl.BlockSpec(shape, lambda *_: tuple(0 for _ in shape))


def kernel(x, edge_index, edge_attr, xA, noiselevel, distances, dosd_distances,
           batch, params):
    p = params
    ei_r = edge_index.astype(jnp.int32)

    def wb(q):
        return [q["w"], q["b"].reshape(1, -1)]

    gnn_ws = []
    for c in ("conv1", "conv2", "conv3"):
        cp = p[c]
        gnn_ws += wb(cp["nn0"]) + wb(cp["nn1"]) + wb(cp["lin_edge"]) + wb(cp["gft"])
        gnn_ws += wb(p["ff" + c[-1]])
    gnn_ws += [p["noise0"]["w"], p["noise0"]["b"].reshape(1, -1),
               p["noise1"]["w"], p["noise1"]["b"].reshape(1, -1)]

    def gnn_wrap(ei_r, x, eattr, xA2, nz2, dosd, *ws):
        f = lambda *refs: _gnn_kernel(*refs[:-1], out_ref=refs[-1])
        return pl.pallas_call(
            f,
            out_shape=jax.ShapeDtypeStruct((N, 256), jnp.float32),
            in_specs=[_full(a.shape) for a in (ei_r, x, eattr, xA2, nz2, dosd)]
            + [_full(w.shape) for w in ws],
            out_specs=_full((N, 256)),
            interpret=INTERPRET,
        )(ei_r, x, eattr, xA2, nz2, dosd, *ws)

    xA2 = xA.reshape(1, NGFEAT)
    nz2 = noiselevel.reshape(1, 1)
    x_base = gnn_wrap(ei_r, x, edge_attr, xA2, nz2, dosd_distances, *gnn_ws)

    def eam_wrap(ei_r, eattr):
        f = lambda a, b, o: _eam_kernel(a, b, out_ref=o)
        return pl.pallas_call(
            f,
            out_shape=jax.ShapeDtypeStruct((N * N, 17), jnp.float32),
            in_specs=[_full(ei_r.shape), _full(eattr.shape)],
            out_specs=_full((N * N, 17)),
            interpret=INTERPRET,
        )(ei_r, eattr)

    eam = eam_wrap(ei_r, edge_attr)

    xann = (jax.nn.relu(xA2 @ p["mg0"]["w"] + p["mg0"]["b"])
            @ p["mg1"]["w"] + p["mg1"]["b"])
    x1 = jnp.broadcast_to(x_base[:, None, :], (N, N, 256)).reshape(N * N, 256)
    x2 = jnp.broadcast_to(x_base[None, :, :], (N, N, 256)).reshape(N * N, 256)
    pf = jnp.concatenate([
        x1, x2,
        jnp.broadcast_to(xann, (N * N, 42)),
        distances.reshape(N * N, 12),
        eam,
        dosd_distances.reshape(N * N, 1),
    ], axis=1)


    pair_ws = []
    for side in ("break", "make"):
        for b in p[side + "_blocks"]:
            pair_ws += [b["ln"]["g"].reshape(1, -1), b["ln"]["b"].reshape(1, -1)]
            pair_ws += wb(b["l0"]) + wb(b["l1"])
    pair_ws += [p["ln_break"]["g"].reshape(1, -1), p["ln_break"]["b"].reshape(1, -1),
                p["ln_make"]["g"].reshape(1, -1), p["ln_make"]["b"].reshape(1, -1)]
    redw = jnp.concatenate([p["red_break"]["w"], p["red_make"]["w"]], axis=1)
    redb = jnp.concatenate([p["red_break"]["b"], p["red_make"]["b"]]).reshape(1, 2)
    pair_ws += [redw, redb]

    T = 512

    def pair_wrap(pf, *ws):
        f = lambda *refs: _pair_kernel(*refs[:-1], out_ref=refs[-1])
        return pl.pallas_call(
            f,
            grid=(N * N // T,),
            out_shape=jax.ShapeDtypeStruct((N * N, 2), jnp.float32),
            in_specs=[pl.BlockSpec((T, D), lambda t: (t, 0))]
            + [pl.BlockSpec(w.shape, functools.partial(lambda t, s: tuple(0 for _ in s), s=w.shape))
               for w in ws],
            out_specs=pl.BlockSpec((T, 2), lambda t: (t, 0)),
            interpret=INTERPRET,
        )(pf, *ws)

    pm_out = pair_wrap(pf, *pair_ws)
    pairs_break = pm_out[:, 0].reshape(1, N, N)
    pairs_make = pm_out[:, 1].reshape(1, N, N)

    pb = pairs_break[0]
    pbs = (pb + pb.T) * 0.5
    pm = pairs_make[0]
    pms = (pm + pm.T) * 0.5

    IB = 8

    def quad_wrap(pbs, pms):
        f = lambda a, b, c, d, o: _quad_kernel(a, b, c, d, out_ref=o)
        return pl.pallas_call(
            f,
            grid=(N // IB,),
            out_shape=jax.ShapeDtypeStruct((N, N, N, N), jnp.float32),
            in_specs=[pl.BlockSpec((IB, N), lambda t: (t, 0)),
                      pl.BlockSpec((IB, N), lambda t: (t, 0)),
                      pl.BlockSpec((N, N), lambda t: (0, 0)),
                      pl.BlockSpec((N, N), lambda t: (0, 0))],
            out_specs=pl.BlockSpec((IB, N, N, N), lambda t: (t, 0, 0, 0)),
            interpret=INTERPRET,
        )(pbs, pms, pbs, pms)

    quad = quad_wrap(pbs, pms)
    return (pairs_break, pairs_make, quad)
